# baseline (speedup 1.0000x reference)
import functools

import jax
import jax.numpy as jnp
from jax import lax
from jax.experimental import pallas as pl
from jax.experimental.pallas import tpu as pltpu
from jax.experimental.pallas import tpu_sc as plsc

B = 4096
IN_DIM = 768
H0 = 512
H1 = 256
LAT = 256
K = 8192
CW = 0.25

BM = 512
GRID = B // BM
BMC = 256
MGRID = B // BMC
NCB = K // GRID


def _f32(*shape):
    return jax.ShapeDtypeStruct(shape, jnp.float32)


def _enc_body(x_ref, w0, b0, w1, b1, w2, b2, c0, c1, c2,
              z_ref, n0_ref, o0, o1, o2, cbn0_ref, cbn2_ref):
    x = x_ref[...]
    h = jnp.maximum(jnp.dot(x, w0[...], preferred_element_type=jnp.float32) + b0[...], 0.0)
    h = jnp.maximum(jnp.dot(h, w1[...], preferred_element_type=jnp.float32) + b1[...], 0.0)
    z = jnp.dot(h, w2[...], preferred_element_type=jnp.float32) + b2[...]
    z_ref[...] = z
    n0_ref[...] = jnp.sqrt(jnp.sum(z * z, axis=1, keepdims=True))
    for c, o, nref in ((c0, o0, cbn0_ref), (c1, o1, None), (c2, o2, cbn2_ref)):
        cv = c[...]
        nrm = jnp.sqrt(jnp.sum(cv * cv, axis=1, keepdims=True))
        o[...] = cv / (nrm + 1e-8)
        if nref is not None:
            nref[...] = nrm


def _encode(x, w0, b0, w1, b1, w2, b2, cb0, cb1, cb2):
    cspec = pl.BlockSpec((NCB, LAT), lambda i: (i, 0))
    nspec = pl.BlockSpec((NCB, 1), lambda i: (i, 0))
    return pl.pallas_call(
        _enc_body,
        grid=(GRID,),
        in_specs=[
            pl.BlockSpec((BM, IN_DIM), lambda i: (i, 0)),
            pl.BlockSpec((IN_DIM, H0), lambda i: (0, 0)),
            pl.BlockSpec((1, H0), lambda i: (0, 0)),
            pl.BlockSpec((H0, H1), lambda i: (0, 0)),
            pl.BlockSpec((1, H1), lambda i: (0, 0)),
            pl.BlockSpec((H1, LAT), lambda i: (0, 0)),
            pl.BlockSpec((1, LAT), lambda i: (0, 0)),
            cspec, cspec, cspec,
        ],
        out_specs=[
            pl.BlockSpec((BM, LAT), lambda i: (i, 0)),
            pl.BlockSpec((BM, 1), lambda i: (i, 0)),
            cspec, cspec, cspec, nspec, nspec,
        ],
        out_shape=[_f32(B, LAT), _f32(B, 1), _f32(K, LAT), _f32(K, LAT),
                   _f32(K, LAT), _f32(K, 1), _f32(K, 1)],
    )(x, w0, b0, w1, b1, w2, b2, cb0, cb1, cb2)


def _argmax_ids(res, cn_ref):
    nrm = jnp.sqrt(jnp.sum(res * res, axis=1, keepdims=True))
    rn = res / (nrm + 1e-8)
    sims = lax.dot_general(rn, cn_ref[...], (((1,), (1,)), ((), ())),
                           preferred_element_type=jnp.float32)
    maxv = jnp.max(sims, axis=1, keepdims=True)
    oh = jnp.where(sims >= maxv, 1.0, 0.0)
    iota_col = lax.broadcasted_iota(jnp.int32, (K, 1), 0).astype(jnp.float32)
    idsf = jnp.dot(oh, iota_col, precision=lax.Precision.HIGHEST,
                   preferred_element_type=jnp.float32)
    return jnp.minimum(idsf.astype(jnp.int32), K - 1)


def _argmax0_body(z_ref, cn_ref, ids_ref):
    ids_ref[...] = _argmax_ids(z_ref[...], cn_ref)


def _argmax0(z, cn):
    return pl.pallas_call(
        _argmax0_body,
        grid=(GRID,),
        in_specs=[
            pl.BlockSpec((BM, LAT), lambda i: (i, 0)),
            pl.BlockSpec((K, LAT), lambda i: (0, 0)),
        ],
        out_specs=pl.BlockSpec((BM, 1), lambda i: (i, 0)),
        out_shape=jax.ShapeDtypeStruct((B, 1), jnp.int32),
    )(z, cn)


def _step_body(rp_ref, emb_ref, cn_ref, ids_ref, res_ref, d_ref, en_ref):
    rp = rp_ref[...]
    emb = emb_ref[...]
    res = rp - emb
    d_ref[...] = jnp.sum(res * res, axis=1, keepdims=True)
    en_ref[...] = jnp.sqrt(jnp.sum(emb * emb, axis=1, keepdims=True))
    res_ref[...] = res
    ids_ref[...] = _argmax_ids(res, cn_ref)


def _step(res_prev, emb_prev, cn):
    return pl.pallas_call(
        _step_body,
        grid=(GRID,),
        in_specs=[
            pl.BlockSpec((BM, LAT), lambda i: (i, 0)),
            pl.BlockSpec((BM, LAT), lambda i: (i, 0)),
            pl.BlockSpec((K, LAT), lambda i: (0, 0)),
        ],
        out_specs=[
            pl.BlockSpec((BM, 1), lambda i: (i, 0)),
            pl.BlockSpec((BM, LAT), lambda i: (i, 0)),
            pl.BlockSpec((BM, 1), lambda i: (i, 0)),
            pl.BlockSpec((BM, 1), lambda i: (i, 0)),
        ],
        out_shape=[jax.ShapeDtypeStruct((B, 1), jnp.int32), _f32(B, LAT),
                   _f32(B, 1), _f32(B, 1)],
    )(res_prev, emb_prev, cn)


def _sc_gather(table, idx):
    info = plsc.get_sparse_core_info()
    num_cores = info.num_cores
    nw = num_cores * info.num_subcores
    bpw = B // nw
    mesh = plsc.VectorSubcoreMesh(core_axis_name="c", subcore_axis_name="s")

    @functools.partial(
        pl.kernel,
        mesh=mesh,
        out_type=jax.ShapeDtypeStruct((B, LAT), jnp.float32),
        scratch_types=[
            pltpu.VMEM((bpw,), jnp.int32),
            pltpu.VMEM((bpw, LAT), jnp.float32),
            pltpu.SemaphoreType.DMA,
        ],
    )
    def gather_kernel(idx_hbm, table_hbm, out_hbm, idx_v, rows_v, sem):
        wid = lax.axis_index("s") * num_cores + lax.axis_index("c")
        base = wid * bpw
        pltpu.sync_copy(idx_hbm.at[pl.ds(base, bpw)], idx_v)
        pltpu.async_copy(table_hbm.at[idx_v], rows_v, sem).wait()
        pltpu.sync_copy(rows_v, out_hbm.at[pl.ds(base, bpw)])

    return gather_kernel(idx, table)


def _dec_body(z_ref, r2_ref, e2_ref, x_ref, w0, b0, w1, b1, w2, b2,
              d2_ref, en2_ref, rec_ref):
    r2 = r2_ref[...]
    e2 = e2_ref[...]
    res3 = r2 - e2
    d2_ref[...] = jnp.sum(res3 * res3, axis=1, keepdims=True)
    en2_ref[...] = jnp.sqrt(jnp.sum(e2 * e2, axis=1, keepdims=True))
    zsum = z_ref[...] - res3
    h = jnp.maximum(jnp.dot(zsum, w0[...], preferred_element_type=jnp.float32) + b0[...], 0.0)
    h = jnp.maximum(jnp.dot(h, w1[...], preferred_element_type=jnp.float32) + b1[...], 0.0)
    xh = jnp.dot(h, w2[...], preferred_element_type=jnp.float32) + b2[...]
    dx = xh - x_ref[...]
    rec_ref[...] = jnp.sum(dx * dx, axis=1, keepdims=True)


def _decode(z, res2, emb2, x, w0, b0, w1, b1, w2, b2):
    return pl.pallas_call(
        _dec_body,
        grid=(GRID,),
        in_specs=[
            pl.BlockSpec((BM, LAT), lambda i: (i, 0)),
            pl.BlockSpec((BM, LAT), lambda i: (i, 0)),
            pl.BlockSpec((BM, LAT), lambda i: (i, 0)),
            pl.BlockSpec((BM, IN_DIM), lambda i: (i, 0)),
            pl.BlockSpec((LAT, H1), lambda i: (0, 0)),
            pl.BlockSpec((1, H1), lambda i: (0, 0)),
            pl.BlockSpec((H1, H0), lambda i: (0, 0)),
            pl.BlockSpec((1, H0), lambda i: (0, 0)),
            pl.BlockSpec((H0, IN_DIM), lambda i: (0, 0)),
            pl.BlockSpec((1, IN_DIM), lambda i: (0, 0)),
        ],
        out_specs=[
            pl.BlockSpec((BM, 1), lambda i: (i, 0)),
            pl.BlockSpec((BM, 1), lambda i: (i, 0)),
            pl.BlockSpec((BM, 1), lambda i: (i, 0)),
        ],
        out_shape=[_f32(B, 1), _f32(B, 1), _f32(B, 1)],
    )(z, res2, emb2, x, w0, b0, w1, b1, w2, b2)


def _metrics_body(i0_t, i1_t, i2_t, i0_f, i1_f, i2_f, en0_t, en1_t, en2_t,
                  n0_ref, d0_ref, d1_ref, d2_ref, rec_ref, cbn0_ref, cbn2_ref,
                  embn_ref, loss_ref, recon_ref, ql_ref, pu_ref, frn_ref,
                  lrn_ref, fcn_ref, lcn_ref, cov_ref, ent_ref,
                  counts_acc, uniq_acc):
    step = pl.program_id(0)

    embn_ref[...] = jnp.concatenate([en0_t[...], en1_t[...], en2_t[...]], axis=1)

    @pl.when(step == 0)
    def _():
        counts_acc[...] = jnp.zeros_like(counts_acc)
        uniq_acc[0, 0] = 0.0

    kio = lax.broadcasted_iota(jnp.int32, (BMC, K), 1)
    for j, ref in enumerate((i0_t, i1_t, i2_t)):
        oh = (ref[...] == kio).astype(jnp.int32)
        counts_acc[j:j + 1, :] += jnp.sum(oh, axis=0, keepdims=True)

    eq = ((i0_t[...] == i0_f[...]) & (i1_t[...] == i1_f[...])
          & (i2_t[...] == i2_f[...]))
    iota = lax.broadcasted_iota(jnp.int32, (BMC, B), 1)
    first = jnp.min(jnp.where(eq, iota, B), axis=1, keepdims=True)
    rows = lax.broadcasted_iota(jnp.int32, (BMC, 1), 0) + step * BMC
    uniq_acc[0, 0] += jnp.sum((first == rows).astype(jnp.float32))

    @pl.when(step == MGRID - 1)
    def _():
        fb = jnp.float32(B)
        ql = (1.0 + CW) * (jnp.sum(d0_ref[...]) + jnp.sum(d1_ref[...])
                           + jnp.sum(d2_ref[...])) / fb
        recon = jnp.sum(rec_ref[...]) / jnp.float32(B * IN_DIM)
        input_norm = jnp.maximum(jnp.sum(n0_ref[...]) / fb, 1e-8)
        full = lambda v: jnp.full((1, 1), v, jnp.float32)
        loss_ref[...] = full(recon + ql)
        recon_ref[...] = full(recon)
        ql_ref[...] = full(ql)
        pu_ref[...] = full(uniq_acc[0, 0] / fb)
        frn_ref[...] = full((jnp.sum(jnp.sqrt(d0_ref[...])) / fb) / input_norm)
        lrn_ref[...] = full((jnp.sum(jnp.sqrt(d2_ref[...])) / fb) / input_norm)
        fcn_ref[...] = full(jnp.sum(cbn0_ref[...]) / jnp.float32(K))
        lcn_ref[...] = full(jnp.sum(cbn2_ref[...]) / jnp.float32(K))
        counts = counts_acc[...].astype(jnp.float32)
        cov_ref[...] = jnp.sum((counts > 0).astype(jnp.float32), axis=1,
                               keepdims=True) / jnp.float32(K)
        probs = counts / jnp.sum(counts, axis=1, keepdims=True)
        plogp = jnp.where(probs > 0,
                          probs * jnp.log(jnp.where(probs > 0, probs, 1.0)), 0.0)
        ent_ref[...] = -jnp.sum(plogp, axis=1, keepdims=True)


def _metrics(ids0, ids1, ids2, en0, en1, en2, n0, d0, d1, d2, rec, cbn0, cbn2):
    tspec = pl.BlockSpec((BMC, 1), lambda i: (i, 0))
    fspec = pl.BlockSpec((1, B), lambda i: (0, 0))
    vspec = pl.BlockSpec((B, 1), lambda i: (0, 0))
    kspec = pl.BlockSpec((K, 1), lambda i: (0, 0))
    sspec = pl.BlockSpec((1, 1), lambda i: (0, 0))
    return pl.pallas_call(
        _metrics_body,
        grid=(MGRID,),
        in_specs=[tspec, tspec, tspec, fspec, fspec, fspec,
                  tspec, tspec, tspec,
                  vspec, vspec, vspec, vspec, vspec, kspec, kspec],
        out_specs=[pl.BlockSpec((BMC, 3), lambda i: (i, 0))]
                  + [sspec] * 8
                  + [pl.BlockSpec((3, 1), lambda i: (0, 0))] * 2,
        out_shape=[_f32(B, 3)] + [_f32(1, 1)] * 8 + [_f32(3, 1), _f32(3, 1)],
        scratch_shapes=[
            pltpu.VMEM((3, K), jnp.int32),
            pltpu.SMEM((1, 1), jnp.float32),
        ],
    )(ids0, ids1, ids2, ids0.reshape(1, B), ids1.reshape(1, B),
      ids2.reshape(1, B), en0, en1, en2, n0, d0, d1, d2, rec, cbn0, cbn2)


def kernel(x, enc_w0, enc_b0, enc_w1, enc_b1, enc_w2, enc_b2,
           dec_w0, dec_b0, dec_w1, dec_b1, dec_w2, dec_b2, cb0, cb1, cb2):
    z, n0, cn0, cn1, cn2, cbn0, cbn2 = _encode(
        x, enc_w0, enc_b0.reshape(1, H0), enc_w1, enc_b1.reshape(1, H1),
        enc_w2, enc_b2.reshape(1, LAT), cb0, cb1, cb2)

    ids0 = _argmax0(z, cn0)
    emb0 = _sc_gather(cb0, ids0.reshape(B))
    ids1, res1, d0, en0 = _step(z, emb0, cn1)
    emb1 = _sc_gather(cb1, ids1.reshape(B))
    ids2, res2, d1, en1 = _step(res1, emb1, cn2)
    emb2 = _sc_gather(cb2, ids2.reshape(B))

    d2, en2, rec = _decode(z, res2, emb2, x, dec_w0, dec_b0.reshape(1, H1),
                           dec_w1, dec_b1.reshape(1, H0), dec_w2,
                           dec_b2.reshape(1, IN_DIM))

    (embs_norm, loss, recon, ql, pu, frn, lrn, fcn, lcn, cov, ent) = _metrics(
        ids0, ids1, ids2, en0, en1, en2, n0, d0, d1, d2, rec, cbn0, cbn2)

    return (loss.reshape(()), recon.reshape(()), ql.reshape(()), embs_norm,
            pu.reshape(()), cov.reshape(3), ent.reshape(3), frn.reshape(()),
            lrn.reshape(()), fcn.reshape(()), lcn.reshape(()))


# speedup vs baseline: 1.7701x; 1.7701x over previous
import functools

import jax
import jax.numpy as jnp
from jax import lax
from jax.experimental import pallas as pl
from jax.experimental.pallas import tpu as pltpu
from jax.experimental.pallas import tpu_sc as plsc

B = 4096
IN_DIM = 768
H0 = 512
H1 = 256
LAT = 256
K = 8192
CW = 0.25

BM = 512
GRID = B // BM
NCB = K // GRID
NCORES = 2
CNT_W = 8


def _f32(*shape):
    return jax.ShapeDtypeStruct(shape, jnp.float32)


def _enc_body(x_ref, w0, b0, w1, b1, w2, b2, c0, c1, c2,
              z_ref, n0_ref, o0, o1, o2, cbn0_ref, cbn2_ref):
    x = x_ref[...]
    h = jnp.maximum(jnp.dot(x, w0[...], preferred_element_type=jnp.float32) + b0[...], 0.0)
    h = jnp.maximum(jnp.dot(h, w1[...], preferred_element_type=jnp.float32) + b1[...], 0.0)
    z = jnp.dot(h, w2[...], preferred_element_type=jnp.float32) + b2[...]
    z_ref[...] = z
    n0_ref[...] = jnp.sqrt(jnp.sum(z * z, axis=1, keepdims=True))
    for c, o, nref in ((c0, o0, cbn0_ref), (c1, o1, None), (c2, o2, cbn2_ref)):
        cv = c[...]
        nrm = jnp.sqrt(jnp.sum(cv * cv, axis=1, keepdims=True))
        o[...] = cv / (nrm + 1e-8)
        if nref is not None:
            nref[...] = nrm


def _encode(x, w0, b0, w1, b1, w2, b2, cb0, cb1, cb2):
    cspec = pl.BlockSpec((NCB, LAT), lambda i: (i, 0))
    nspec = pl.BlockSpec((NCB, 1), lambda i: (i, 0))
    return pl.pallas_call(
        _enc_body,
        grid=(GRID,),
        in_specs=[
            pl.BlockSpec((BM, IN_DIM), lambda i: (i, 0)),
            pl.BlockSpec((IN_DIM, H0), lambda i: (0, 0)),
            pl.BlockSpec((1, H0), lambda i: (0, 0)),
            pl.BlockSpec((H0, H1), lambda i: (0, 0)),
            pl.BlockSpec((1, H1), lambda i: (0, 0)),
            pl.BlockSpec((H1, LAT), lambda i: (0, 0)),
            pl.BlockSpec((1, LAT), lambda i: (0, 0)),
            cspec, cspec, cspec,
        ],
        out_specs=[
            pl.BlockSpec((BM, LAT), lambda i: (i, 0)),
            pl.BlockSpec((BM, 1), lambda i: (i, 0)),
            cspec, cspec, cspec, nspec, nspec,
        ],
        out_shape=[_f32(B, LAT), _f32(B, 1), _f32(K, LAT), _f32(K, LAT),
                   _f32(K, LAT), _f32(K, 1), _f32(K, 1)],
    )(x, w0, b0, w1, b1, w2, b2, cb0, cb1, cb2)


def _argmax_ids(res, cn_ref, ids_ref, cnt_ref):
    nrm = jnp.sqrt(jnp.sum(res * res, axis=1, keepdims=True))
    rn = res / (nrm + 1e-8)
    sims = lax.dot_general(rn, cn_ref[...], (((1,), (1,)), ((), ())),
                           preferred_element_type=jnp.float32)
    maxv = jnp.max(sims, axis=1, keepdims=True)
    oh = jnp.where(sims >= maxv, 1.0, 0.0)
    io = lax.broadcasted_iota(jnp.int32, (K, 2), 0)
    hilo = jnp.where(lax.broadcasted_iota(jnp.int32, (K, 2), 1) == 0,
                     io // 128, io % 128).astype(jnp.float32)
    hl = jnp.dot(oh, hilo, preferred_element_type=jnp.float32)
    ids = (hl[:, 0:1] * 128.0 + hl[:, 1:2]).astype(jnp.int32)
    ids_ref[...] = jnp.clip(ids, 0, K - 1)

    @pl.when(pl.program_id(0) == 0)
    def _():
        cnt_ref[...] = jnp.zeros_like(cnt_ref)

    cnt_ref[...] += jnp.dot(jnp.ones((1, BM), jnp.float32), oh,
                            preferred_element_type=jnp.float32)


def _argmax0_body(z_ref, cn_ref, ids_ref, cnt_ref):
    _argmax_ids(z_ref[...], cn_ref, ids_ref, cnt_ref)


def _argmax0(z, cn):
    return pl.pallas_call(
        _argmax0_body,
        grid=(GRID,),
        in_specs=[
            pl.BlockSpec((BM, LAT), lambda i: (i, 0)),
            pl.BlockSpec((K, LAT), lambda i: (0, 0)),
        ],
        out_specs=[
            pl.BlockSpec((BM, 1), lambda i: (i, 0)),
            pl.BlockSpec((1, K), lambda i: (0, 0)),
        ],
        out_shape=[jax.ShapeDtypeStruct((B, 1), jnp.int32), _f32(1, K)],
    )(z, cn)


def _step_body(rp_ref, emb_ref, cn_ref, ids_ref, cnt_ref, res_ref, d_ref,
               en_ref):
    rp = rp_ref[...]
    emb = emb_ref[...]
    res = rp - emb
    d_ref[...] = jnp.sum(res * res, axis=1, keepdims=True)
    en_ref[...] = jnp.sqrt(jnp.sum(emb * emb, axis=1, keepdims=True))
    res_ref[...] = res
    _argmax_ids(res, cn_ref, ids_ref, cnt_ref)


def _step(res_prev, emb_prev, cn):
    return pl.pallas_call(
        _step_body,
        grid=(GRID,),
        in_specs=[
            pl.BlockSpec((BM, LAT), lambda i: (i, 0)),
            pl.BlockSpec((BM, LAT), lambda i: (i, 0)),
            pl.BlockSpec((K, LAT), lambda i: (0, 0)),
        ],
        out_specs=[
            pl.BlockSpec((BM, 1), lambda i: (i, 0)),
            pl.BlockSpec((1, K), lambda i: (0, 0)),
            pl.BlockSpec((BM, LAT), lambda i: (i, 0)),
            pl.BlockSpec((BM, 1), lambda i: (i, 0)),
            pl.BlockSpec((BM, 1), lambda i: (i, 0)),
        ],
        out_shape=[jax.ShapeDtypeStruct((B, 1), jnp.int32), _f32(1, K),
                   _f32(B, LAT), _f32(B, 1), _f32(B, 1)],
    )(res_prev, emb_prev, cn)


def _sc_gather(table, idx):
    info = plsc.get_sparse_core_info()
    num_cores = info.num_cores
    nw = num_cores * info.num_subcores
    bpw = B // nw
    mesh = plsc.VectorSubcoreMesh(core_axis_name="c", subcore_axis_name="s")

    @functools.partial(
        pl.kernel,
        mesh=mesh,
        out_type=jax.ShapeDtypeStruct((B, LAT), jnp.float32),
        scratch_types=[
            pltpu.VMEM((bpw,), jnp.int32),
            pltpu.VMEM((bpw, LAT), jnp.float32),
            pltpu.SemaphoreType.DMA,
        ],
    )
    def gather_kernel(idx_hbm, table_hbm, out_hbm, idx_v, rows_v, sem):
        wid = lax.axis_index("s") * num_cores + lax.axis_index("c")
        base = wid * bpw
        pltpu.sync_copy(idx_hbm.at[pl.ds(base, bpw)], idx_v)
        pltpu.async_copy(table_hbm.at[idx_v], rows_v, sem).wait()
        pltpu.sync_copy(rows_v, out_hbm.at[pl.ds(base, bpw)])

    return gather_kernel(idx, table)


def _dec_body(z_ref, r2_ref, e2_ref, x_ref, w0, b0, w1, b1, w2, b2,
              d2_ref, en2_ref, rec_ref):
    r2 = r2_ref[...]
    e2 = e2_ref[...]
    res3 = r2 - e2
    d2_ref[...] = jnp.sum(res3 * res3, axis=1, keepdims=True)
    en2_ref[...] = jnp.sqrt(jnp.sum(e2 * e2, axis=1, keepdims=True))
    zsum = z_ref[...] - res3
    h = jnp.maximum(jnp.dot(zsum, w0[...], preferred_element_type=jnp.float32) + b0[...], 0.0)
    h = jnp.maximum(jnp.dot(h, w1[...], preferred_element_type=jnp.float32) + b1[...], 0.0)
    xh = jnp.dot(h, w2[...], preferred_element_type=jnp.float32) + b2[...]
    dx = xh - x_ref[...]
    rec_ref[...] = jnp.sum(dx * dx, axis=1, keepdims=True)


def _decode(z, res2, emb2, x, w0, b0, w1, b1, w2, b2):
    return pl.pallas_call(
        _dec_body,
        grid=(GRID,),
        in_specs=[
            pl.BlockSpec((BM, LAT), lambda i: (i, 0)),
            pl.BlockSpec((BM, LAT), lambda i: (i, 0)),
            pl.BlockSpec((BM, LAT), lambda i: (i, 0)),
            pl.BlockSpec((BM, IN_DIM), lambda i: (i, 0)),
            pl.BlockSpec((LAT, H1), lambda i: (0, 0)),
            pl.BlockSpec((1, H1), lambda i: (0, 0)),
            pl.BlockSpec((H1, H0), lambda i: (0, 0)),
            pl.BlockSpec((1, H0), lambda i: (0, 0)),
            pl.BlockSpec((H0, IN_DIM), lambda i: (0, 0)),
            pl.BlockSpec((1, IN_DIM), lambda i: (0, 0)),
        ],
        out_specs=[
            pl.BlockSpec((BM, 1), lambda i: (i, 0)),
            pl.BlockSpec((BM, 1), lambda i: (i, 0)),
            pl.BlockSpec((BM, 1), lambda i: (i, 0)),
        ],
        out_shape=[_f32(B, 1), _f32(B, 1), _f32(B, 1)],
    )(z, res2, emb2, x, w0, b0, w1, b1, w2, b2)


def _uniq_body(a_t, b_t, c_t, a_f, b_f, c_f, u_ref):
    i = pl.program_id(0)
    eq = ((a_t[...] == a_f[...]) & (b_t[...] == b_f[...]) & (c_t[...] == c_f[...]))
    iota = lax.broadcasted_iota(jnp.int32, (BM, B), 1)
    first = jnp.min(jnp.where(eq, iota, B), axis=1, keepdims=True)
    rows = lax.broadcasted_iota(jnp.int32, (BM, 1), 0) + i * BM
    u_ref[...] = (first == rows).astype(jnp.float32)


def _unique_flags(ids0, ids1, ids2):
    tspec = pl.BlockSpec((BM, 1), lambda i: (i, 0))
    fspec = pl.BlockSpec((1, B), lambda i: (0, 0))
    return pl.pallas_call(
        _uniq_body,
        grid=(GRID,),
        in_specs=[tspec, tspec, tspec, fspec, fspec, fspec],
        out_specs=pl.BlockSpec((BM, 1), lambda i: (i, 0)),
        out_shape=_f32(B, 1),
    )(ids0, ids1, ids2, ids0.reshape(1, B), ids1.reshape(1, B),
      ids2.reshape(1, B))


def _scalars_body(n0_ref, d0_ref, d1_ref, d2_ref, rec_ref, uniq_ref,
                  c0_ref, c1_ref, c2_ref, cbn0_ref, cbn2_ref,
                  sv_ref, cov_ref, ent_ref):
    fb = jnp.float32(B)
    ql = (1.0 + CW) * (jnp.sum(d0_ref[...]) + jnp.sum(d1_ref[...])
                       + jnp.sum(d2_ref[...])) / fb
    recon = jnp.sum(rec_ref[...]) / jnp.float32(B * IN_DIM)
    loss = recon + ql
    p_unique = jnp.sum(uniq_ref[...]) / fb
    input_norm = jnp.maximum(jnp.sum(n0_ref[...]) / fb, 1e-8)
    first_rn = (jnp.sum(jnp.sqrt(d0_ref[...])) / fb) / input_norm
    last_rn = (jnp.sum(jnp.sqrt(d2_ref[...])) / fb) / input_norm
    first_cn = jnp.sum(cbn0_ref[...]) / jnp.float32(K)
    last_cn = jnp.sum(cbn2_ref[...]) / jnp.float32(K)

    covs = []
    ents = []
    for cref in (c0_ref, c1_ref, c2_ref):
        cnt = cref[...]
        covs.append(jnp.sum((cnt > 0).astype(jnp.float32)) / jnp.float32(K))
        probs = cnt / jnp.sum(cnt)
        plogp = jnp.where(probs > 0,
                          probs * jnp.log(jnp.where(probs > 0, probs, 1.0)), 0.0)
        ents.append(-jnp.sum(plogp))

    io3 = lax.broadcasted_iota(jnp.int32, (3, 1), 0)
    cov_ref[...] = jnp.where(io3 == 0, covs[0],
                             jnp.where(io3 == 1, covs[1], covs[2]))
    ent_ref[...] = jnp.where(io3 == 0, ents[0],
                             jnp.where(io3 == 1, ents[1], ents[2]))

    io = lax.broadcasted_iota(jnp.int32, (8, 1), 0)
    sv = jnp.where(io == 0, loss,
         jnp.where(io == 1, recon,
         jnp.where(io == 2, ql,
         jnp.where(io == 3, p_unique,
         jnp.where(io == 4, first_rn,
         jnp.where(io == 5, last_rn,
         jnp.where(io == 6, first_cn, last_cn)))))))
    sv_ref[...] = sv


def _scalars(n0, d0, d1, d2, rec, uniq, c0, c1, c2, cbn0, cbn2):
    whole = lambda s: pl.BlockSpec(s, lambda: tuple(0 for _ in s))
    return pl.pallas_call(
        _scalars_body,
        in_specs=[whole((B, 1))] * 6 + [whole((1, K))] * 3
                 + [whole((K, 1)), whole((K, 1))],
        out_specs=[whole((8, 1)), whole((3, 1)), whole((3, 1))],
        out_shape=[_f32(8, 1), _f32(3, 1), _f32(3, 1)],
    )(n0, d0, d1, d2, rec, uniq, c0, c1, c2, cbn0, cbn2)


def kernel(x, enc_w0, enc_b0, enc_w1, enc_b1, enc_w2, enc_b2,
           dec_w0, dec_b0, dec_w1, dec_b1, dec_w2, dec_b2, cb0, cb1, cb2):
    z, n0, cn0, cn1, cn2, cbn0, cbn2 = _encode(
        x, enc_w0, enc_b0.reshape(1, H0), enc_w1, enc_b1.reshape(1, H1),
        enc_w2, enc_b2.reshape(1, LAT), cb0, cb1, cb2)

    ids0, cnt0 = _argmax0(z, cn0)
    emb0 = _sc_gather(cb0, ids0.reshape(B))
    ids1, cnt1, res1, d0, en0 = _step(z, emb0, cn1)
    emb1 = _sc_gather(cb1, ids1.reshape(B))
    ids2, cnt2, res2, d1, en1 = _step(res1, emb1, cn2)
    emb2 = _sc_gather(cb2, ids2.reshape(B))

    d2, en2, rec = _decode(z, res2, emb2, x, dec_w0, dec_b0.reshape(1, H1),
                           dec_w1, dec_b1.reshape(1, H0), dec_w2,
                           dec_b2.reshape(1, IN_DIM))

    uniq = _unique_flags(ids0, ids1, ids2)
    sv, cov, ent = _scalars(n0, d0, d1, d2, rec, uniq, cnt0, cnt1, cnt2,
                            cbn0, cbn2)

    s = sv.reshape(8)
    embs_norm = jnp.concatenate([en0, en1, en2], axis=1)
    return (s[0], s[1], s[2], embs_norm, s[3], cov.reshape(3), ent.reshape(3),
            s[4], s[5], s[6], s[7])


# speedup vs baseline: 2.3310x; 1.3169x over previous
import functools

import jax
import jax.numpy as jnp
from jax import lax
from jax.experimental import pallas as pl
from jax.experimental.pallas import tpu as pltpu
from jax.experimental.pallas import tpu_sc as plsc

B = 4096
IN_DIM = 768
H0 = 512
H1 = 256
LAT = 256
K = 8192
CW = 0.25

BM = 512
GRID = B // BM
NCB = K // GRID
NCORES = 2
CNT_W = 8


def _f32(*shape):
    return jax.ShapeDtypeStruct(shape, jnp.float32)


def _enc_body(x_ref, w0, b0, w1, b1, w2, b2, c0, c1, c2,
              z_ref, n0_ref, o0, o1, o2, cbn0_ref, cbn2_ref):
    x = x_ref[...]
    h = jnp.maximum(jnp.dot(x, w0[...], preferred_element_type=jnp.float32) + b0[...], 0.0)
    h = jnp.maximum(jnp.dot(h, w1[...], preferred_element_type=jnp.float32) + b1[...], 0.0)
    z = jnp.dot(h, w2[...], preferred_element_type=jnp.float32) + b2[...]
    z_ref[...] = z
    n0_ref[...] = jnp.sqrt(jnp.sum(z * z, axis=1, keepdims=True))
    for c, o, nref in ((c0, o0, cbn0_ref), (c1, o1, None), (c2, o2, cbn2_ref)):
        cv = c[...]
        nrm = jnp.sqrt(jnp.sum(cv * cv, axis=1, keepdims=True))
        o[...] = cv / (nrm + 1e-8)
        if nref is not None:
            nref[...] = nrm


def _encode(x, w0, b0, w1, b1, w2, b2, cb0, cb1, cb2):
    cspec = pl.BlockSpec((NCB, LAT), lambda i: (i, 0))
    nspec = pl.BlockSpec((NCB, 1), lambda i: (i, 0))
    return pl.pallas_call(
        _enc_body,
        grid=(GRID,),
        in_specs=[
            pl.BlockSpec((BM, IN_DIM), lambda i: (i, 0)),
            pl.BlockSpec((IN_DIM, H0), lambda i: (0, 0)),
            pl.BlockSpec((1, H0), lambda i: (0, 0)),
            pl.BlockSpec((H0, H1), lambda i: (0, 0)),
            pl.BlockSpec((1, H1), lambda i: (0, 0)),
            pl.BlockSpec((H1, LAT), lambda i: (0, 0)),
            pl.BlockSpec((1, LAT), lambda i: (0, 0)),
            cspec, cspec, cspec,
        ],
        out_specs=[
            pl.BlockSpec((BM, LAT), lambda i: (i, 0)),
            pl.BlockSpec((BM, 1), lambda i: (i, 0)),
            cspec, cspec, cspec, nspec, nspec,
        ],
        out_shape=[_f32(B, LAT), _f32(B, 1), _f32(K, LAT), _f32(K, LAT),
                   _f32(K, LAT), _f32(K, 1), _f32(K, 1)],
    )(x, w0, b0, w1, b1, w2, b2, cb0, cb1, cb2)


def _argmax_ids(res, cn_ref, ids_ref, cnt_ref):
    nrm = jnp.sqrt(jnp.sum(res * res, axis=1, keepdims=True))
    rn = res / (nrm + 1e-8)
    sims = lax.dot_general(rn, cn_ref[...], (((1,), (1,)), ((), ())),
                           preferred_element_type=jnp.float32)
    maxv = jnp.max(sims, axis=1, keepdims=True)
    hit = sims >= maxv
    iota = lax.broadcasted_iota(jnp.int32, sims.shape, 1)
    ids_ref[...] = jnp.min(jnp.where(hit, iota, K), axis=1, keepdims=True)

    @pl.when(pl.program_id(0) == 0)
    def _():
        cnt_ref[...] = jnp.zeros_like(cnt_ref)

    oh = jnp.where(hit, 1.0, 0.0)
    cnt_ref[...] += jnp.dot(jnp.ones((1, BM), jnp.float32), oh,
                            preferred_element_type=jnp.float32)


def _argmax0_body(z_ref, cn_ref, ids_ref, cnt_ref):
    _argmax_ids(z_ref[...], cn_ref, ids_ref, cnt_ref)


def _argmax0(z, cn):
    return pl.pallas_call(
        _argmax0_body,
        grid=(GRID,),
        in_specs=[
            pl.BlockSpec((BM, LAT), lambda i: (i, 0)),
            pl.BlockSpec((K, LAT), lambda i: (0, 0)),
        ],
        out_specs=[
            pl.BlockSpec((BM, 1), lambda i: (i, 0)),
            pl.BlockSpec((1, K), lambda i: (0, 0)),
        ],
        out_shape=[jax.ShapeDtypeStruct((B, 1), jnp.int32), _f32(1, K)],
    )(z, cn)


def _step_body(rp_ref, emb_ref, cn_ref, ids_ref, cnt_ref, res_ref, d_ref,
               en_ref):
    rp = rp_ref[...]
    emb = emb_ref[...]
    res = rp - emb
    d_ref[...] = jnp.sum(res * res, axis=1, keepdims=True)
    en_ref[...] = jnp.sqrt(jnp.sum(emb * emb, axis=1, keepdims=True))
    res_ref[...] = res
    _argmax_ids(res, cn_ref, ids_ref, cnt_ref)


def _step(res_prev, emb_prev, cn):
    return pl.pallas_call(
        _step_body,
        grid=(GRID,),
        in_specs=[
            pl.BlockSpec((BM, LAT), lambda i: (i, 0)),
            pl.BlockSpec((BM, LAT), lambda i: (i, 0)),
            pl.BlockSpec((K, LAT), lambda i: (0, 0)),
        ],
        out_specs=[
            pl.BlockSpec((BM, 1), lambda i: (i, 0)),
            pl.BlockSpec((1, K), lambda i: (0, 0)),
            pl.BlockSpec((BM, LAT), lambda i: (i, 0)),
            pl.BlockSpec((BM, 1), lambda i: (i, 0)),
            pl.BlockSpec((BM, 1), lambda i: (i, 0)),
        ],
        out_shape=[jax.ShapeDtypeStruct((B, 1), jnp.int32), _f32(1, K),
                   _f32(B, LAT), _f32(B, 1), _f32(B, 1)],
    )(res_prev, emb_prev, cn)


def _sc_gather(table, idx):
    info = plsc.get_sparse_core_info()
    num_cores = info.num_cores
    nw = num_cores * info.num_subcores
    bpw = B // nw
    mesh = plsc.VectorSubcoreMesh(core_axis_name="c", subcore_axis_name="s")

    @functools.partial(
        pl.kernel,
        mesh=mesh,
        out_type=jax.ShapeDtypeStruct((B, LAT), jnp.float32),
        scratch_types=[
            pltpu.VMEM((bpw,), jnp.int32),
            pltpu.VMEM((bpw, LAT), jnp.float32),
            pltpu.SemaphoreType.DMA,
        ],
    )
    def gather_kernel(idx_hbm, table_hbm, out_hbm, idx_v, rows_v, sem):
        wid = lax.axis_index("s") * num_cores + lax.axis_index("c")
        base = wid * bpw
        pltpu.sync_copy(idx_hbm.at[pl.ds(base, bpw)], idx_v)
        pltpu.async_copy(table_hbm.at[idx_v], rows_v, sem).wait()
        pltpu.sync_copy(rows_v, out_hbm.at[pl.ds(base, bpw)])

    return gather_kernel(idx, table)


def _dec_body(z_ref, r2_ref, e2_ref, x_ref, w0, b0, w1, b1, w2, b2,
              d2_ref, en2_ref, rec_ref):
    r2 = r2_ref[...]
    e2 = e2_ref[...]
    res3 = r2 - e2
    d2_ref[...] = jnp.sum(res3 * res3, axis=1, keepdims=True)
    en2_ref[...] = jnp.sqrt(jnp.sum(e2 * e2, axis=1, keepdims=True))
    zsum = z_ref[...] - res3
    h = jnp.maximum(jnp.dot(zsum, w0[...], preferred_element_type=jnp.float32) + b0[...], 0.0)
    h = jnp.maximum(jnp.dot(h, w1[...], preferred_element_type=jnp.float32) + b1[...], 0.0)
    xh = jnp.dot(h, w2[...], preferred_element_type=jnp.float32) + b2[...]
    dx = xh - x_ref[...]
    rec_ref[...] = jnp.sum(dx * dx, axis=1, keepdims=True)


def _decode(z, res2, emb2, x, w0, b0, w1, b1, w2, b2):
    return pl.pallas_call(
        _dec_body,
        grid=(GRID,),
        in_specs=[
            pl.BlockSpec((BM, LAT), lambda i: (i, 0)),
            pl.BlockSpec((BM, LAT), lambda i: (i, 0)),
            pl.BlockSpec((BM, LAT), lambda i: (i, 0)),
            pl.BlockSpec((BM, IN_DIM), lambda i: (i, 0)),
            pl.BlockSpec((LAT, H1), lambda i: (0, 0)),
            pl.BlockSpec((1, H1), lambda i: (0, 0)),
            pl.BlockSpec((H1, H0), lambda i: (0, 0)),
            pl.BlockSpec((1, H0), lambda i: (0, 0)),
            pl.BlockSpec((H0, IN_DIM), lambda i: (0, 0)),
            pl.BlockSpec((1, IN_DIM), lambda i: (0, 0)),
        ],
        out_specs=[
            pl.BlockSpec((BM, 1), lambda i: (i, 0)),
            pl.BlockSpec((BM, 1), lambda i: (i, 0)),
            pl.BlockSpec((BM, 1), lambda i: (i, 0)),
        ],
        out_shape=[_f32(B, 1), _f32(B, 1), _f32(B, 1)],
    )(z, res2, emb2, x, w0, b0, w1, b1, w2, b2)


def _uniq_body(a_t, b_t, c_t, a_f, b_f, c_f, u_ref):
    i = pl.program_id(0)
    eq = ((a_t[...] == a_f[...]) & (b_t[...] == b_f[...]) & (c_t[...] == c_f[...]))
    iota = lax.broadcasted_iota(jnp.int32, (BM, B), 1)
    first = jnp.min(jnp.where(eq, iota, B), axis=1, keepdims=True)
    rows = lax.broadcasted_iota(jnp.int32, (BM, 1), 0) + i * BM
    u_ref[...] = (first == rows).astype(jnp.float32)


def _unique_flags(ids0, ids1, ids2):
    tspec = pl.BlockSpec((BM, 1), lambda i: (i, 0))
    fspec = pl.BlockSpec((1, B), lambda i: (0, 0))
    return pl.pallas_call(
        _uniq_body,
        grid=(GRID,),
        in_specs=[tspec, tspec, tspec, fspec, fspec, fspec],
        out_specs=pl.BlockSpec((BM, 1), lambda i: (i, 0)),
        out_shape=_f32(B, 1),
    )(ids0, ids1, ids2, ids0.reshape(1, B), ids1.reshape(1, B),
      ids2.reshape(1, B))


def _scalars_body(n0_ref, d0_ref, d1_ref, d2_ref, rec_ref, uniq_ref,
                  c0_ref, c1_ref, c2_ref, cbn0_ref, cbn2_ref,
                  sv_ref, cov_ref, ent_ref):
    fb = jnp.float32(B)
    ql = (1.0 + CW) * (jnp.sum(d0_ref[...]) + jnp.sum(d1_ref[...])
                       + jnp.sum(d2_ref[...])) / fb
    recon = jnp.sum(rec_ref[...]) / jnp.float32(B * IN_DIM)
    loss = recon + ql
    p_unique = jnp.sum(uniq_ref[...]) / fb
    input_norm = jnp.maximum(jnp.sum(n0_ref[...]) / fb, 1e-8)
    first_rn = (jnp.sum(jnp.sqrt(d0_ref[...])) / fb) / input_norm
    last_rn = (jnp.sum(jnp.sqrt(d2_ref[...])) / fb) / input_norm
    first_cn = jnp.sum(cbn0_ref[...]) / jnp.float32(K)
    last_cn = jnp.sum(cbn2_ref[...]) / jnp.float32(K)

    covs = []
    ents = []
    for cref in (c0_ref, c1_ref, c2_ref):
        cnt = cref[...]
        covs.append(jnp.sum((cnt > 0).astype(jnp.float32)) / jnp.float32(K))
        probs = cnt / jnp.sum(cnt)
        plogp = jnp.where(probs > 0,
                          probs * jnp.log(jnp.where(probs > 0, probs, 1.0)), 0.0)
        ents.append(-jnp.sum(plogp))

    io3 = lax.broadcasted_iota(jnp.int32, (3, 1), 0)
    cov_ref[...] = jnp.where(io3 == 0, covs[0],
                             jnp.where(io3 == 1, covs[1], covs[2]))
    ent_ref[...] = jnp.where(io3 == 0, ents[0],
                             jnp.where(io3 == 1, ents[1], ents[2]))

    io = lax.broadcasted_iota(jnp.int32, (8, 1), 0)
    sv = jnp.where(io == 0, loss,
         jnp.where(io == 1, recon,
         jnp.where(io == 2, ql,
         jnp.where(io == 3, p_unique,
         jnp.where(io == 4, first_rn,
         jnp.where(io == 5, last_rn,
         jnp.where(io == 6, first_cn, last_cn)))))))
    sv_ref[...] = sv


def _scalars(n0, d0, d1, d2, rec, uniq, c0, c1, c2, cbn0, cbn2):
    whole = lambda s: pl.BlockSpec(s, lambda: tuple(0 for _ in s))
    return pl.pallas_call(
        _scalars_body,
        in_specs=[whole((B, 1))] * 6 + [whole((1, K))] * 3
                 + [whole((K, 1)), whole((K, 1))],
        out_specs=[whole((8, 1)), whole((3, 1)), whole((3, 1))],
        out_shape=[_f32(8, 1), _f32(3, 1), _f32(3, 1)],
    )(n0, d0, d1, d2, rec, uniq, c0, c1, c2, cbn0, cbn2)


def kernel(x, enc_w0, enc_b0, enc_w1, enc_b1, enc_w2, enc_b2,
           dec_w0, dec_b0, dec_w1, dec_b1, dec_w2, dec_b2, cb0, cb1, cb2):
    z, n0, cn0, cn1, cn2, cbn0, cbn2 = _encode(
        x, enc_w0, enc_b0.reshape(1, H0), enc_w1, enc_b1.reshape(1, H1),
        enc_w2, enc_b2.reshape(1, LAT), cb0, cb1, cb2)

    ids0, cnt0 = _argmax0(z, cn0)
    emb0 = _sc_gather(cb0, ids0.reshape(B))
    ids1, cnt1, res1, d0, en0 = _step(z, emb0, cn1)
    emb1 = _sc_gather(cb1, ids1.reshape(B))
    ids2, cnt2, res2, d1, en1 = _step(res1, emb1, cn2)
    emb2 = _sc_gather(cb2, ids2.reshape(B))

    d2, en2, rec = _decode(z, res2, emb2, x, dec_w0, dec_b0.reshape(1, H1),
                           dec_w1, dec_b1.reshape(1, H0), dec_w2,
                           dec_b2.reshape(1, IN_DIM))

    uniq = _unique_flags(ids0, ids1, ids2)
    sv, cov, ent = _scalars(n0, d0, d1, d2, rec, uniq, cnt0, cnt1, cnt2,
                            cbn0, cbn2)

    s = sv.reshape(8)
    embs_norm = jnp.concatenate([en0, en1, en2], axis=1)
    return (s[0], s[1], s[2], embs_norm, s[3], cov.reshape(3), ent.reshape(3),
            s[4], s[5], s[6], s[7])
